# Initial kernel scaffold; baseline (speedup 1.0000x reference)
import jax
import jax.numpy as jnp
from jax.experimental import pallas as pl


def kernel(x1, h1, x2, h2, W1, b1, gamma, beta, W2, b2):
    raise NotImplementedError("write your pallas kernel here")


# speedup vs baseline: 5.4070x; 5.4070x over previous
import functools

import jax
import jax.numpy as jnp
import numpy as np
from jax import lax
from jax.experimental import pallas as pl
from jax.experimental.pallas import tpu as pltpu
from jax.experimental.pallas import tpu_sc as plsc

H = 128
N = 8192
N1 = 4096
K = 10
D2 = 2 * H
DW = 384
NEG_SLOPE = 0.02
EPS = 1e-5

DST_BLK = 256
N_DST_BLK = N // DST_BLK
SRC_CHUNK = 256
N_SRC_CHUNK = N // SRC_CHUNK
KPAD = 16

IDX_MASK = np.int32(0x1FFF)
KEY_MASK = np.int32(~0x1FFF)
KEY_INF = np.int32(0x7FFFFFFF)

ROW_BLK = 512
N_ROW_BLK = N // ROW_BLK


def _knn_body(xcol_ref, xrow_ref, out_ref, key_ref):
    sqr = (xrow_ref[0:1, :] * xrow_ref[0:1, :]
           + xrow_ref[1:2, :] * xrow_ref[1:2, :]
           + xrow_ref[2:3, :] * xrow_ref[2:3, :])

    def build(ci, _):
        sl = pl.ds(ci * SRC_CHUNK, SRC_CHUNK)
        sqc = (xcol_ref[sl, 0:1] * xcol_ref[sl, 0:1]
               + xcol_ref[sl, 1:2] * xcol_ref[sl, 1:2]
               + xcol_ref[sl, 2:3] * xcol_ref[sl, 2:3])
        dot = jnp.dot(xcol_ref[sl, :], xrow_ref[...],
                      preferred_element_type=jnp.float32)
        d = (sqc + sqr) - 2.0 * dot
        d = jnp.maximum(d, 0.0)
        srcid = lax.broadcasted_iota(jnp.int32, (SRC_CHUNK, DST_BLK), 0)
        srcid = srcid + ci * SRC_CHUNK
        key = (lax.bitcast_convert_type(d, jnp.int32) & KEY_MASK) | srcid
        key_ref[sl, :] = key
        return 0

    lax.fori_loop(0, N_SRC_CHUNK, build, 0)

    m_prev = jnp.full((1, DST_BLK), -1, jnp.int32)
    for k in range(K):
        def scan(ci, acc, m_prev=m_prev):
            sl = pl.ds(ci * SRC_CHUNK, SRC_CHUNK)
            ch = key_ref[sl, :]
            ch = jnp.where(ch > m_prev, ch, KEY_INF)
            return jnp.minimum(acc, jnp.min(ch, axis=0, keepdims=True))

        m_prev = lax.fori_loop(0, N_SRC_CHUNK, scan,
                               jnp.full((1, DST_BLK), KEY_INF, jnp.int32))
        out_ref[k:k + 1, :] = m_prev & IDX_MASK


def _knn_call(xpad, xt):
    return pl.pallas_call(
        _knn_body,
        grid=(N_DST_BLK,),
        in_specs=[
            pl.BlockSpec((N, 8), lambda i: (0, 0)),
            pl.BlockSpec((8, DST_BLK), lambda i: (0, i)),
        ],
        out_specs=pl.BlockSpec((KPAD, DST_BLK), lambda i: (0, i)),
        out_shape=jax.ShapeDtypeStruct((KPAD, N), jnp.int32),
        scratch_shapes=[pltpu.VMEM((N, DST_BLK), jnp.int32)],
    )(xpad, xt)


def _proj_body(h_ref, w_ref, b1_ref, x_ref, a_ref, b_ref):
    hw = jnp.dot(h_ref[...], w_ref[...], preferred_element_type=jnp.float32)
    zpad = jnp.zeros((ROW_BLK, DW - D2 - 9), jnp.float32)
    ones1 = jnp.ones((ROW_BLK, 1), jnp.float32)
    a_ref[...] = jnp.concatenate(
        [hw[:, :D2], x_ref[...], ones1, zpad], axis=1)
    b_ref[...] = jnp.concatenate(
        [hw[:, D2:] + b1_ref[...], x_ref[...], 0.0 * ones1, zpad], axis=1)


def _proj_call(h, wcat, b1r, xpad):
    return pl.pallas_call(
        _proj_body,
        grid=(N_ROW_BLK,),
        in_specs=[
            pl.BlockSpec((ROW_BLK, H), lambda i: (i, 0)),
            pl.BlockSpec((H, 2 * D2), lambda i: (0, 0)),
            pl.BlockSpec((1, D2), lambda i: (0, 0)),
            pl.BlockSpec((ROW_BLK, 8), lambda i: (i, 0)),
        ],
        out_specs=[
            pl.BlockSpec((ROW_BLK, DW), lambda i: (i, 0)),
            pl.BlockSpec((ROW_BLK, DW), lambda i: (i, 0)),
        ],
        out_shape=[
            jax.ShapeDtypeStruct((N, DW), jnp.float32),
            jax.ShapeDtypeStruct((N, DW), jnp.float32),
        ],
    )(h, wcat, b1r, xpad)


_SC_WORKERS = 32
_GCHUNK = 128


def _gather_body(idx_hbm, tab_hbm, out_hbm, idx_v, rows_v, sem):
    wid = lax.axis_index("s") * 2 + lax.axis_index("c")
    for k in range(K):
        for half in range(DST_BLK // _GCHUNK):
            base = k * N + wid * DST_BLK + half * _GCHUNK
            pltpu.sync_copy(idx_hbm.at[pl.ds(base, _GCHUNK)], idx_v)
            pltpu.async_copy(tab_hbm.at[idx_v], rows_v, sem).wait()
            pltpu.sync_copy(rows_v, out_hbm.at[pl.ds(base, _GCHUNK)])


def _gather_call(idx_flat, a_ext):
    mesh = plsc.VectorSubcoreMesh(
        core_axis_name="c", subcore_axis_name="s", num_cores=2)
    fn = pl.kernel(
        _gather_body,
        mesh=mesh,
        out_type=jax.ShapeDtypeStruct((K * N, DW), jnp.float32),
        scratch_types=[
            pltpu.VMEM((_GCHUNK,), jnp.int32),
            pltpu.VMEM((_GCHUNK, DW), jnp.float32),
            pltpu.SemaphoreType.DMA,
        ],
    )
    return fn(idx_flat, a_ext)


def _edge_body(g_ref, b_ref, c_ref, out_ref):
    g = g_ref[...]
    b = b_ref[...]
    s = g + b
    t = jnp.where(s >= 0.0, s, NEG_SLOPE * s)
    r = jnp.sum(t * c_ref[...], axis=1, keepdims=True)
    xd = g[:, D2:D2 + 8] - b[:, D2:D2 + 8]
    contrib = (r * (1.0 / K)) * xd
    k = pl.program_id(1)

    @pl.when(k == 0)
    def _():
        out_ref[...] = b[:, D2:D2 + 8] + contrib

    @pl.when(k != 0)
    def _():
        out_ref[...] = out_ref[...] + contrib


def _edge_call(g, b_ext, cvec):
    return pl.pallas_call(
        _edge_body,
        grid=(N_DST_BLK, K),
        in_specs=[
            pl.BlockSpec((DST_BLK, DW), lambda i, k: (k * N_DST_BLK + i, 0)),
            pl.BlockSpec((DST_BLK, DW), lambda i, k: (i, 0)),
            pl.BlockSpec((1, DW), lambda i, k: (0, 0)),
        ],
        out_specs=pl.BlockSpec((DST_BLK, 8), lambda i, k: (i, 0)),
        out_shape=jax.ShapeDtypeStruct((N, 8), jnp.float32),
    )(g, b_ext, cvec)


def kernel(x1, h1, x2, h2, W1, b1, gamma, beta, W2, b2):
    x = jnp.concatenate([x1, x2], axis=0)
    h = jnp.concatenate([h1, h2], axis=0)
    xpad = jnp.pad(x, ((0, 0), (0, 5)))
    xt = xpad.T

    idx16 = _knn_call(xpad, xt)
    idx_flat = idx16[:K, :].reshape(-1)

    wcat = jnp.concatenate([W1[:, :H].T, W1[:, H:].T], axis=1)
    a_ext, b_ext = _proj_call(h, wcat, b1.reshape(1, D2), xpad)

    g = _gather_call(idx_flat, a_ext)

    scale = np.float32(1.0 / np.sqrt(1.0 + EPS))
    c256 = (gamma * scale) * W2[0]
    c0 = jnp.dot(beta, W2[0]) + b2[0]
    cvec = jnp.concatenate(
        [c256, jnp.zeros((8,), jnp.float32), c0[None],
         jnp.zeros((DW - D2 - 9,), jnp.float32)]).reshape(1, DW)

    out8 = _edge_call(g, b_ext, cvec)
    xn = out8[:, :3]
    return (xn[:N1], xn[N1:])
